# baseline (speedup 1.0000x reference)
import functools
import jax
import jax.numpy as jnp
from jax import lax
from jax.experimental import pallas as pl
from jax.experimental.pallas import tpu as pltpu
from jax.experimental.pallas import tpu_sc as plsc

N = 10000
NPAD = 10240
E = 320000
K = 128
NC, NS = 2, 16
NW = NC * NS
CPW = 80
HB = 40
NCHUNK = NW * CPW
EPAD = NCHUNK * K
ROWS_PER_SUB = NPAD // NS


def _sc_body(with_deg, x_hbm, src_hbm, dst_hbm, zrow_hbm, zone_hbm,
             *refs):
    if with_deg:
        (acc_out, deg_out, src_l, dst_l, rows0, rows1, ones_v,
         semg0, semg1, sems0, sems1, acc_sh, deg_sh) = refs
    else:
        (acc_out, src_l, dst_l, rows0, rows1,
         semg0, semg1, sems0, sems1, acc_sh) = refs
    c = lax.axis_index("c")
    s = lax.axis_index("s")
    w = s * NC + c

    pltpu.sync_copy(zrow_hbm, acc_sh.at[pl.ds(s * ROWS_PER_SUB, ROWS_PER_SUB)])
    if with_deg:
        pltpu.sync_copy(zone_hbm, deg_sh.at[pl.ds(s * ROWS_PER_SUB, ROWS_PER_SUB)])
        for j in range(K // 16):
            ones_v[pl.ds(j * 16, 16)] = jnp.ones((16,), jnp.float32)
    plsc.subcore_barrier()

    def gather(t, rows, sem):
        return pltpu.make_async_copy(x_hbm.at[src_l.at[t, 0]], rows, sem)

    class _Scatter:
        def __init__(self, t, rows, sem):
            self.t, self.rows, self.sem = t, rows, sem

        def start(self):
            pltpu.async_copy(self.rows, acc_sh.at[dst_l.at[self.t, 0]],
                             self.sem, add=True)

        def wait(self):
            pltpu.make_async_copy(self.rows, acc_sh.at[dst_l.at[self.t, 0]],
                                  self.sem).wait()

    scatter = _Scatter

    def deg_scatter(t):
        if with_deg:
            pltpu.sync_copy(ones_v, deg_sh.at[dst_l.at[t, 0]], add=True)

    def chunk(t, carry):
        gather(t, rows0, semg0).start()
        gather(t, rows0, semg0).wait()
        pltpu.sync_copy(rows0, acc_sh.at[dst_l.at[t, 0]], add=True)
        deg_scatter(t)
        return carry

    for h in range(CPW // HB):
        base = w * CPW + h * HB
        pltpu.sync_copy(src_hbm.at[pl.ds(base, HB)], src_l)
        pltpu.sync_copy(dst_hbm.at[pl.ds(base, HB)], dst_l)
        lax.fori_loop(0, HB, chunk, 0)
    plsc.subcore_barrier()

    sl = pl.ds(s * ROWS_PER_SUB, ROWS_PER_SUB)
    pltpu.sync_copy(acc_sh.at[sl], acc_out.at[c, sl])
    if with_deg:
        pltpu.sync_copy(deg_sh.at[sl], deg_out.at[c, sl])


def _make_sc_call(with_deg):
    out_type = [jax.ShapeDtypeStruct((NC, NPAD, 128), jnp.float32)]
    scratch = [
        pltpu.VMEM((HB, 1, K), jnp.int32),
        pltpu.VMEM((HB, 1, K), jnp.int32),
        pltpu.VMEM((K, 128), jnp.float32),
        pltpu.VMEM((K, 128), jnp.float32),
    ]
    if with_deg:
        out_type.append(jax.ShapeDtypeStruct((NC, NPAD), jnp.float32))
        scratch.append(pltpu.VMEM((K,), jnp.float32))
    scratch.extend([pltpu.SemaphoreType.DMA] * 4)
    scratch.append(pltpu.VMEM_SHARED((NPAD, 128), jnp.float32))
    if with_deg:
        scratch.append(pltpu.VMEM_SHARED((NPAD,), jnp.float32))
    mesh = plsc.VectorSubcoreMesh(core_axis_name="c", subcore_axis_name="s",
                                  num_cores=NC, num_subcores=NS)
    return pl.kernel(
        functools.partial(_sc_body, with_deg),
        out_type=tuple(out_type),
        mesh=mesh,
        scratch_types=tuple(scratch),
        name="sage_segsum_sc" + ("_deg" if with_deg else ""),
    )


def _dense_body(a0, a1, d0, d1, xb, WlT, bl, WrT, out):
    deg = jnp.maximum(d0[...] + d1[...], 1.0)
    agg = (a0[...] + a1[...]) / deg
    h = (jnp.dot(agg, WlT[...], preferred_element_type=jnp.float32)
         + bl[...]
         + jnp.dot(xb[...], WrT[...], preferred_element_type=jnp.float32))
    out[...] = jnp.maximum(h, 0.0)


BR = 1280


def _dense_call(a0, a1, d0, d1, xb, WlT, bl, WrT):
    nblk = NPAD // BR
    row = lambda i: (i, 0)
    fixed = lambda i: (0, 0)
    return pl.pallas_call(
        _dense_body,
        grid=(nblk,),
        in_specs=[
            pl.BlockSpec((BR, 128), row),
            pl.BlockSpec((BR, 128), row),
            pl.BlockSpec((BR, 1), row),
            pl.BlockSpec((BR, 1), row),
            pl.BlockSpec((BR, 128), row),
            pl.BlockSpec((128, 128), fixed),
            pl.BlockSpec((1, 128), fixed),
            pl.BlockSpec((128, 128), fixed),
        ],
        out_specs=pl.BlockSpec((BR, 128), row),
        out_shape=jax.ShapeDtypeStruct((NPAD, 128), jnp.float32),
    )(a0, a1, d0, d1, xb, WlT, bl, WrT)


def _tail_body(h2, batch2d, set01, WmdT, WmmT, WmxT, bm, W1T, b1,
               W2Tp, b2p, out):
    b = batch2d[...]
    g3 = lax.broadcasted_iota(jnp.int32, (128, 80, 128), 0)
    cmp = (b[None, :, :] < g3).astype(jnp.int32)
    base = jnp.sum(jnp.sum(cmp, axis=2), axis=1, keepdims=True)
    idx0 = jnp.clip(base + set01[:, 0:1], 0, N - 1)
    idx1 = jnp.clip(base + set01[:, 1:2], 0, N - 1)
    col = lax.broadcasted_iota(jnp.int32, (128, NPAD), 1)
    h = h2[...]
    xs0 = jnp.dot((col == idx0).astype(jnp.float32), h,
                  preferred_element_type=jnp.float32)
    xs1 = jnp.dot((col == idx1).astype(jnp.float32), h,
                  preferred_element_type=jnp.float32)
    d = jnp.abs(xs0 - xs1)
    m = (xs0 + xs1) * 0.5
    x = jnp.maximum(xs0, xs1)
    pooled = (jnp.dot(d, WmdT[...], preferred_element_type=jnp.float32)
              + jnp.dot(m, WmmT[...], preferred_element_type=jnp.float32)
              + jnp.dot(x, WmxT[...], preferred_element_type=jnp.float32)
              + bm[...])
    f = jnp.maximum(
        jnp.dot(pooled, W1T[...], preferred_element_type=jnp.float32) + b1[...],
        0.0)
    logits = jnp.dot(f, W2Tp[...], preferred_element_type=jnp.float32) + b2p[...]
    mx = jnp.max(logits, axis=1, keepdims=True)
    lse = jnp.log(jnp.sum(jnp.exp(logits - mx), axis=1, keepdims=True))
    out[...] = logits - mx - lse


def _tail_call(h2, batch2d, set01, WmdT, WmmT, WmxT, bm, W1T, b1, W2Tp, b2p):
    return pl.pallas_call(
        _tail_body,
        out_shape=jax.ShapeDtypeStruct((128, 128), jnp.float32),
    )(h2, batch2d, set01, WmdT, WmmT, WmxT, bm, W1T, b1, W2Tp, b2p)


def kernel(x, edge_index, set_indices, batch, num_graphs,
           Wl1, bl1, Wr1, Wl2, bl2, Wr2, Wm, bm, W1, b1, W2, b2):
    del num_graphs
    f32 = jnp.float32

    xp = jnp.pad(x, ((0, NPAD - N), (0, 0)))
    src2d = jnp.pad(edge_index[0], (0, EPAD - E)).reshape(NCHUNK, 1, K)
    dst2d = jnp.pad(edge_index[1], (0, EPAD - E),
                    constant_values=NPAD - 1).reshape(NCHUNK, 1, K)
    zrow = jnp.zeros((ROWS_PER_SUB, 128), f32)
    zone = jnp.zeros((ROWS_PER_SUB,), f32)
    batch2d = jnp.pad(batch, (0, NPAD - N), constant_values=127).reshape(80, 128)
    set01 = jnp.pad(set_indices, ((0, 128 - set_indices.shape[0]), (0, 6)))
    Wl1T, Wr1T = Wl1.T, Wr1.T
    Wl2T, Wr2T = Wl2.T, Wr2.T
    bl1r, bl2r = bl1.reshape(1, 128), bl2.reshape(1, 128)
    WmdT = Wm[:, 0:128].T
    WmmT = Wm[:, 128:256].T
    WmxT = Wm[:, 256:384].T
    bmr = bm.reshape(1, 128)
    W1T = W1.T
    b1r = b1.reshape(1, 128)
    W2Tp = jnp.pad(W2.T, ((0, 0), (0, 128 - W2.shape[0])))
    b2p = jnp.pad(b2, (0, 128 - W2.shape[0]),
                  constant_values=-1e30).reshape(1, 128)

    acc1, deg = _make_sc_call(True)(xp, src2d, dst2d, zrow, zone)
    d0 = deg[0].reshape(NPAD, 1)
    d1 = deg[1].reshape(NPAD, 1)
    h1 = _dense_call(acc1[0], acc1[1], d0, d1, xp, Wl1T, bl1r, Wr1T)

    acc2 = _make_sc_call(False)(h1, src2d, dst2d, zrow, zone)[0]
    h2 = _dense_call(acc2[0], acc2[1], d0, d1, h1, Wl2T, bl2r, Wr2T)

    outp = _tail_call(h2, batch2d, set01, WmdT, WmmT, WmxT, bmr,
                      W1T, b1r, W2Tp, b2p)
    return outp[:set_indices.shape[0], :W2.shape[0]]


# speedup vs baseline: 1.0661x; 1.0661x over previous
import functools
import jax
import jax.numpy as jnp
from jax import lax
from jax.experimental import pallas as pl
from jax.experimental.pallas import tpu as pltpu
from jax.experimental.pallas import tpu_sc as plsc

N = 10000
NPAD = 10240
E = 320000
K = 128
NC, NS = 2, 16
NW = NC * NS
CPW = 80
HB = 40
NCHUNK = NW * CPW
EPAD = NCHUNK * K
ROWS_PER_SUB = NPAD // NS


def _sc_body(with_deg, x_hbm, src_hbm, dst_hbm, zrow_hbm, zone_hbm,
             *refs):
    if with_deg:
        (acc_out, deg_out, idx_s, idx_d, rows0, ones_v,
         semg0, acc_sh, deg_sh) = refs
    else:
        (acc_out, idx_s, idx_d, rows0, semg0, acc_sh) = refs
    c = lax.axis_index("c")
    s = lax.axis_index("s")
    w = s * NC + c

    pltpu.sync_copy(zrow_hbm, acc_sh.at[pl.ds(s * ROWS_PER_SUB, ROWS_PER_SUB)])
    if with_deg:
        pltpu.sync_copy(zone_hbm, deg_sh.at[pl.ds(s * ROWS_PER_SUB, ROWS_PER_SUB)])
        for j in range(K // 16):
            ones_v[pl.ds(j * 16, 16)] = jnp.ones((16,), jnp.float32)
    plsc.subcore_barrier()

    def chunk(t, carry):
        cid = w + NW * t
        pltpu.sync_copy(src_hbm.at[cid], idx_s)
        pltpu.sync_copy(dst_hbm.at[cid], idx_d)
        pltpu.async_copy(x_hbm.at[idx_s], rows0, semg0).wait()
        pltpu.sync_copy(rows0, acc_sh.at[idx_d], add=True)
        if with_deg:
            pltpu.sync_copy(ones_v, deg_sh.at[idx_d], add=True)
        return carry

    lax.fori_loop(0, CPW, chunk, 0)
    plsc.subcore_barrier()

    sl = pl.ds(s * ROWS_PER_SUB, ROWS_PER_SUB)
    pltpu.sync_copy(acc_sh.at[sl], acc_out.at[c, sl])
    if with_deg:
        pltpu.sync_copy(deg_sh.at[sl], deg_out.at[c, sl])


def _make_sc_call(with_deg):
    out_type = [jax.ShapeDtypeStruct((NC, NPAD, 128), jnp.float32)]
    scratch = [
        pltpu.VMEM((K,), jnp.int32),
        pltpu.VMEM((K,), jnp.int32),
        pltpu.VMEM((K, 128), jnp.float32),
    ]
    if with_deg:
        out_type.append(jax.ShapeDtypeStruct((NC, NPAD), jnp.float32))
        scratch.append(pltpu.VMEM((K,), jnp.float32))
    scratch.append(pltpu.SemaphoreType.DMA)
    scratch.append(pltpu.VMEM_SHARED((NPAD, 128), jnp.float32))
    if with_deg:
        scratch.append(pltpu.VMEM_SHARED((NPAD,), jnp.float32))
    mesh = plsc.VectorSubcoreMesh(core_axis_name="c", subcore_axis_name="s",
                                  num_cores=NC, num_subcores=NS)
    return pl.kernel(
        functools.partial(_sc_body, with_deg),
        out_type=tuple(out_type),
        mesh=mesh,
        scratch_types=tuple(scratch),
        name="sage_segsum_sc" + ("_deg" if with_deg else ""),
    )


def _dense_body(a0, a1, d0, d1, xb, WlT, bl, WrT, out):
    deg = jnp.maximum(d0[...] + d1[...], 1.0)
    agg = (a0[...] + a1[...]) / deg
    h = (jnp.dot(agg, WlT[...], preferred_element_type=jnp.float32)
         + bl[...]
         + jnp.dot(xb[...], WrT[...], preferred_element_type=jnp.float32))
    out[...] = jnp.maximum(h, 0.0)


BR = 1280


def _dense_call(a0, a1, d0, d1, xb, WlT, bl, WrT):
    nblk = NPAD // BR
    row = lambda i: (i, 0)
    fixed = lambda i: (0, 0)
    return pl.pallas_call(
        _dense_body,
        grid=(nblk,),
        in_specs=[
            pl.BlockSpec((BR, 128), row),
            pl.BlockSpec((BR, 128), row),
            pl.BlockSpec((BR, 1), row),
            pl.BlockSpec((BR, 1), row),
            pl.BlockSpec((BR, 128), row),
            pl.BlockSpec((128, 128), fixed),
            pl.BlockSpec((1, 128), fixed),
            pl.BlockSpec((128, 128), fixed),
        ],
        out_specs=pl.BlockSpec((BR, 128), row),
        out_shape=jax.ShapeDtypeStruct((NPAD, 128), jnp.float32),
    )(a0, a1, d0, d1, xb, WlT, bl, WrT)


def _tail_body(h2, batch2d, set01, WmdT, WmmT, WmxT, bm, W1T, b1,
               W2Tp, b2p, out):
    b = batch2d[...]
    g3 = lax.broadcasted_iota(jnp.int32, (128, 80, 128), 0)
    cmp = (b[None, :, :] < g3).astype(jnp.int32)
    base = jnp.sum(jnp.sum(cmp, axis=2), axis=1, keepdims=True)
    idx0 = jnp.clip(base + set01[:, 0:1], 0, N - 1)
    idx1 = jnp.clip(base + set01[:, 1:2], 0, N - 1)
    col = lax.broadcasted_iota(jnp.int32, (128, NPAD), 1)
    h = h2[...]
    xs0 = jnp.dot((col == idx0).astype(jnp.float32), h,
                  preferred_element_type=jnp.float32)
    xs1 = jnp.dot((col == idx1).astype(jnp.float32), h,
                  preferred_element_type=jnp.float32)
    d = jnp.abs(xs0 - xs1)
    m = (xs0 + xs1) * 0.5
    x = jnp.maximum(xs0, xs1)
    pooled = (jnp.dot(d, WmdT[...], preferred_element_type=jnp.float32)
              + jnp.dot(m, WmmT[...], preferred_element_type=jnp.float32)
              + jnp.dot(x, WmxT[...], preferred_element_type=jnp.float32)
              + bm[...])
    f = jnp.maximum(
        jnp.dot(pooled, W1T[...], preferred_element_type=jnp.float32) + b1[...],
        0.0)
    logits = jnp.dot(f, W2Tp[...], preferred_element_type=jnp.float32) + b2p[...]
    mx = jnp.max(logits, axis=1, keepdims=True)
    lse = jnp.log(jnp.sum(jnp.exp(logits - mx), axis=1, keepdims=True))
    out[...] = logits - mx - lse


def _tail_call(h2, batch2d, set01, WmdT, WmmT, WmxT, bm, W1T, b1, W2Tp, b2p):
    return pl.pallas_call(
        _tail_body,
        out_shape=jax.ShapeDtypeStruct((128, 128), jnp.float32),
    )(h2, batch2d, set01, WmdT, WmmT, WmxT, bm, W1T, b1, W2Tp, b2p)


def kernel(x, edge_index, set_indices, batch, num_graphs,
           Wl1, bl1, Wr1, Wl2, bl2, Wr2, Wm, bm, W1, b1, W2, b2):
    del num_graphs
    f32 = jnp.float32

    xp = jnp.pad(x, ((0, NPAD - N), (0, 0)))
    src2d = jnp.pad(edge_index[0], (0, EPAD - E)).reshape(NCHUNK, K)
    dst2d = jnp.pad(edge_index[1], (0, EPAD - E),
                    constant_values=NPAD - 1).reshape(NCHUNK, K)
    zrow = jnp.zeros((ROWS_PER_SUB, 128), f32)
    zone = jnp.zeros((ROWS_PER_SUB,), f32)
    batch2d = jnp.pad(batch, (0, NPAD - N), constant_values=127).reshape(80, 128)
    set01 = jnp.pad(set_indices, ((0, 128 - set_indices.shape[0]), (0, 6)))
    Wl1T, Wr1T = Wl1.T, Wr1.T
    Wl2T, Wr2T = Wl2.T, Wr2.T
    bl1r, bl2r = bl1.reshape(1, 128), bl2.reshape(1, 128)
    WmdT = Wm[:, 0:128].T
    WmmT = Wm[:, 128:256].T
    WmxT = Wm[:, 256:384].T
    bmr = bm.reshape(1, 128)
    W1T = W1.T
    b1r = b1.reshape(1, 128)
    W2Tp = jnp.pad(W2.T, ((0, 0), (0, 128 - W2.shape[0])))
    b2p = jnp.pad(b2, (0, 128 - W2.shape[0]),
                  constant_values=-1e30).reshape(1, 128)

    acc1, deg = _make_sc_call(True)(xp, src2d, dst2d, zrow, zone)
    d0 = deg[0].reshape(NPAD, 1)
    d1 = deg[1].reshape(NPAD, 1)
    h1 = _dense_call(acc1[0], acc1[1], d0, d1, xp, Wl1T, bl1r, Wr1T)

    acc2 = _make_sc_call(False)(h1, src2d, dst2d, zrow, zone)[0]
    h2 = _dense_call(acc2[0], acc2[1], d0, d1, h1, Wl2T, bl2r, Wr2T)

    outp = _tail_call(h2, batch2d, set01, WmdT, WmmT, WmxT, bmr,
                      W1T, b1r, W2Tp, b2p)
    return outp[:set_indices.shape[0], :W2.shape[0]]


# speedup vs baseline: 2.0301x; 1.9042x over previous
import functools
import jax
import jax.numpy as jnp
from jax import lax
from jax.experimental import pallas as pl
from jax.experimental.pallas import tpu as pltpu
from jax.experimental.pallas import tpu_sc as plsc

N = 10000
NPAD = 10240
E = 320000
K = 128
NC, NS = 2, 16
NW = NC * NS
CPW = 80
HB = 40
NCHUNK = NW * CPW
EPAD = NCHUNK * K
ROWS_PER_SUB = NPAD // NS


def _sc_body(with_deg, x_hbm, src_hbm, dst_hbm, zrow_hbm, zone_hbm,
             *refs):
    if with_deg:
        (acc_out, deg_out, idx_s, idx_d, rows0, ones_v,
         semg0, acc_sh, deg_sh) = refs
    else:
        (acc_out, idx_s, idx_d, rows0, semg0, acc_sh) = refs
    c = lax.axis_index("c")
    s = lax.axis_index("s")
    w = s * NC + c

    pltpu.sync_copy(zrow_hbm, acc_sh.at[pl.ds(s * ROWS_PER_SUB, ROWS_PER_SUB)])
    if with_deg:
        pltpu.sync_copy(zone_hbm, deg_sh.at[pl.ds(s * ROWS_PER_SUB, ROWS_PER_SUB)])
        for j in range(K // 16):
            ones_v[pl.ds(j * 16, 16)] = jnp.ones((16,), jnp.float32)
    plsc.subcore_barrier()

    def chunk(t, carry):
        cid = w + NW * t
        pltpu.sync_copy(src_hbm.at[cid], idx_s)
        pltpu.sync_copy(dst_hbm.at[cid], idx_d)
        pltpu.async_copy(x_hbm.at[idx_s], rows0, semg0).wait()
        pltpu.sync_copy(rows0, acc_sh.at[idx_d], add=True)
        if with_deg:
            pltpu.sync_copy(ones_v, deg_sh.at[idx_d], add=True)
        return carry

    lax.fori_loop(0, CPW, chunk, 0)
    plsc.subcore_barrier()

    sl = pl.ds(s * ROWS_PER_SUB, ROWS_PER_SUB)
    pltpu.sync_copy(acc_sh.at[sl], acc_out.at[c, sl])
    if with_deg:
        pltpu.sync_copy(deg_sh.at[sl], deg_out.at[c, sl])


def _make_sc_call(with_deg):
    out_type = [jax.ShapeDtypeStruct((NC, NPAD, 128), jnp.float32)]
    scratch = [
        pltpu.VMEM((K,), jnp.int32),
        pltpu.VMEM((K,), jnp.int32),
        pltpu.VMEM((K, 128), jnp.float32),
    ]
    if with_deg:
        out_type.append(jax.ShapeDtypeStruct((NC, NPAD), jnp.float32))
        scratch.append(pltpu.VMEM((K,), jnp.float32))
    scratch.append(pltpu.SemaphoreType.DMA)
    scratch.append(pltpu.VMEM_SHARED((NPAD, 128), jnp.float32))
    if with_deg:
        scratch.append(pltpu.VMEM_SHARED((NPAD,), jnp.float32))
    mesh = plsc.VectorSubcoreMesh(core_axis_name="c", subcore_axis_name="s",
                                  num_cores=NC, num_subcores=NS)
    return pl.kernel(
        functools.partial(_sc_body, with_deg),
        out_type=tuple(out_type),
        mesh=mesh,
        scratch_types=tuple(scratch),
        name="sage_segsum_sc" + ("_deg" if with_deg else ""),
    )


def _dense_body(a0, a1, d0, d1, xb, WlT, bl, WrT, out):
    deg = jnp.maximum(d0[...] + d1[...], 1.0)
    agg = (a0[...] + a1[...]) / deg
    h = (jnp.dot(agg, WlT[...], preferred_element_type=jnp.float32)
         + bl[...]
         + jnp.dot(xb[...], WrT[...], preferred_element_type=jnp.float32))
    out[...] = jnp.maximum(h, 0.0)


BR = 1280


def _dense_call(a0, a1, d0, d1, xb, WlT, bl, WrT):
    nblk = NPAD // BR
    row = lambda i: (i, 0)
    fixed = lambda i: (0, 0)
    return pl.pallas_call(
        _dense_body,
        grid=(nblk,),
        in_specs=[
            pl.BlockSpec((BR, 128), row),
            pl.BlockSpec((BR, 128), row),
            pl.BlockSpec((BR, 1), row),
            pl.BlockSpec((BR, 1), row),
            pl.BlockSpec((BR, 128), row),
            pl.BlockSpec((128, 128), fixed),
            pl.BlockSpec((1, 128), fixed),
            pl.BlockSpec((128, 128), fixed),
        ],
        out_specs=pl.BlockSpec((BR, 128), row),
        out_shape=jax.ShapeDtypeStruct((NPAD, 128), jnp.float32),
    )(a0, a1, d0, d1, xb, WlT, bl, WrT)


def _tail_body(h2, batch2d, set01, WmdT, WmmT, WmxT, bm, W1T, b1,
               W2Tp, b2p, out):
    b = batch2d[...]
    g3 = lax.broadcasted_iota(jnp.int32, (128, 80, 128), 0)
    cmp = (b[None, :, :] < g3).astype(jnp.int32)
    base = jnp.sum(jnp.sum(cmp, axis=2), axis=1, keepdims=True)
    idx0 = jnp.clip(base + set01[:, 0:1], 0, N - 1)
    idx1 = jnp.clip(base + set01[:, 1:2], 0, N - 1)
    col = lax.broadcasted_iota(jnp.int32, (128, NPAD), 1)
    h = h2[...]
    xs0 = jnp.dot((col == idx0).astype(jnp.float32), h,
                  preferred_element_type=jnp.float32)
    xs1 = jnp.dot((col == idx1).astype(jnp.float32), h,
                  preferred_element_type=jnp.float32)
    d = jnp.abs(xs0 - xs1)
    m = (xs0 + xs1) * 0.5
    x = jnp.maximum(xs0, xs1)
    pooled = (jnp.dot(d, WmdT[...], preferred_element_type=jnp.float32)
              + jnp.dot(m, WmmT[...], preferred_element_type=jnp.float32)
              + jnp.dot(x, WmxT[...], preferred_element_type=jnp.float32)
              + bm[...])
    f = jnp.maximum(
        jnp.dot(pooled, W1T[...], preferred_element_type=jnp.float32) + b1[...],
        0.0)
    logits = jnp.dot(f, W2Tp[...], preferred_element_type=jnp.float32) + b2p[...]
    mx = jnp.max(logits, axis=1, keepdims=True)
    lse = jnp.log(jnp.sum(jnp.exp(logits - mx), axis=1, keepdims=True))
    out[...] = logits - mx - lse


def _tail_call(h2, batch2d, set01, WmdT, WmmT, WmxT, bm, W1T, b1, W2Tp, b2p):
    return pl.pallas_call(
        _tail_body,
        out_shape=jax.ShapeDtypeStruct((128, 128), jnp.float32),
    )(h2, batch2d, set01, WmdT, WmmT, WmxT, bm, W1T, b1, W2Tp, b2p)


def kernel(x, edge_index, set_indices, batch, num_graphs,
           Wl1, bl1, Wr1, Wl2, bl2, Wr2, Wm, bm, W1, b1, W2, b2):
    del num_graphs
    f32 = jnp.float32

    xp = jnp.pad(x, ((0, NPAD - N), (0, 0)))
    pad_iota = jnp.arange(EPAD - E, dtype=jnp.int32)
    src2d = jnp.concatenate(
        [edge_index[0], pad_iota % N]).reshape(NCHUNK, K)
    dst2d = jnp.concatenate(
        [edge_index[1], N + pad_iota % (NPAD - N)]).reshape(NCHUNK, K)
    zrow = jnp.zeros((ROWS_PER_SUB, 128), f32)
    zone = jnp.zeros((ROWS_PER_SUB,), f32)
    batch2d = jnp.pad(batch, (0, NPAD - N), constant_values=127).reshape(80, 128)
    set01 = jnp.pad(set_indices, ((0, 128 - set_indices.shape[0]), (0, 6)))
    Wl1T, Wr1T = Wl1.T, Wr1.T
    Wl2T, Wr2T = Wl2.T, Wr2.T
    bl1r, bl2r = bl1.reshape(1, 128), bl2.reshape(1, 128)
    WmdT = Wm[:, 0:128].T
    WmmT = Wm[:, 128:256].T
    WmxT = Wm[:, 256:384].T
    bmr = bm.reshape(1, 128)
    W1T = W1.T
    b1r = b1.reshape(1, 128)
    W2Tp = jnp.pad(W2.T, ((0, 0), (0, 128 - W2.shape[0])))
    b2p = jnp.pad(b2, (0, 128 - W2.shape[0]),
                  constant_values=-1e30).reshape(1, 128)

    acc1, deg = _make_sc_call(True)(xp, src2d, dst2d, zrow, zone)
    d0 = deg[0].reshape(NPAD, 1)
    d1 = deg[1].reshape(NPAD, 1)
    h1 = _dense_call(acc1[0], acc1[1], d0, d1, xp, Wl1T, bl1r, Wr1T)

    acc2 = _make_sc_call(False)(h1, src2d, dst2d, zrow, zone)[0]
    h2 = _dense_call(acc2[0], acc2[1], d0, d1, h1, Wl2T, bl2r, Wr2T)

    outp = _tail_call(h2, batch2d, set01, WmdT, WmmT, WmxT, bmr,
                      W1T, b1r, W2Tp, b2p)
    return outp[:set_indices.shape[0], :W2.shape[0]]


# speedup vs baseline: 2.4141x; 1.1891x over previous
import functools
import jax
import jax.numpy as jnp
from jax import lax
from jax.experimental import pallas as pl
from jax.experimental.pallas import tpu as pltpu
from jax.experimental.pallas import tpu_sc as plsc

N = 10000
NPAD = 10240
E = 320000
K = 128
NC, NS = 2, 16
NW = NC * NS
CPW = 80
HB = 40
NCHUNK = NW * CPW
EPAD = NCHUNK * K
ROWS_PER_SUB = NPAD // NS


def _sc_body(with_deg, x_hbm, src_hbm, dst_hbm, zrow_hbm, zone_hbm,
             *refs):
    if with_deg:
        (acc_out, deg_out, idx_s0, idx_d0, rows0, idx_s1, idx_d1, rows1,
         ones_v, semg0, sems0, semg1, sems1, acc_sh, deg_sh) = refs
    else:
        (acc_out, idx_s0, idx_d0, rows0, idx_s1, idx_d1, rows1,
         semg0, sems0, semg1, sems1, acc_sh) = refs
    c = lax.axis_index("c")
    s = lax.axis_index("s")
    w = s * NC + c

    pltpu.sync_copy(zrow_hbm, acc_sh.at[pl.ds(s * ROWS_PER_SUB, ROWS_PER_SUB)])
    if with_deg:
        pltpu.sync_copy(zone_hbm, deg_sh.at[pl.ds(s * ROWS_PER_SUB, ROWS_PER_SUB)])
        for j in range(K // 16):
            ones_v[pl.ds(j * 16, 16)] = jnp.ones((16,), jnp.float32)
    plsc.subcore_barrier()

    bufs = ((idx_s0, idx_d0, rows0, semg0, sems0),
            (idx_s1, idx_d1, rows1, semg1, sems1))

    def load_idx(t, b):
        cid = w + NW * t
        pltpu.sync_copy(src_hbm.at[cid], bufs[b][0])
        pltpu.sync_copy(dst_hbm.at[cid], bufs[b][1])

    def gather(b):
        return pltpu.make_async_copy(x_hbm.at[bufs[b][0]], bufs[b][2],
                                     bufs[b][3])

    def scatter_start(b):
        pltpu.async_copy(bufs[b][2], acc_sh.at[bufs[b][1]], bufs[b][4],
                         add=True)
        if with_deg:
            pltpu.sync_copy(ones_v, deg_sh.at[bufs[b][1]], add=True)

    def scatter_wait(b):
        pltpu.make_async_copy(bufs[b][2], acc_sh.at[bufs[b][1]],
                              bufs[b][4]).wait()

    def step(tcur, bcur, tnext):
        bnext = 1 - bcur
        gather(bcur).wait()
        scatter_start(bcur)

        @pl.when(tnext < CPW)
        def _():
            @pl.when(tnext - 2 >= 0)
            def _():
                scatter_wait(bnext)
            load_idx(tnext, bnext)
            gather(bnext).start()

    def pair(i, carry):
        t0 = 2 * i
        step(t0, 0, t0 + 1)
        step(t0 + 1, 1, t0 + 2)
        return carry

    load_idx(0, 0)
    gather(0).start()
    lax.fori_loop(0, CPW // 2, pair, 0)
    scatter_wait(0)
    scatter_wait(1)
    plsc.subcore_barrier()

    sl = pl.ds(s * ROWS_PER_SUB, ROWS_PER_SUB)
    pltpu.sync_copy(acc_sh.at[sl], acc_out.at[c, sl])
    if with_deg:
        pltpu.sync_copy(deg_sh.at[sl], deg_out.at[c, sl])


def _make_sc_call(with_deg):
    out_type = [jax.ShapeDtypeStruct((NC, NPAD, 128), jnp.float32)]
    scratch = [
        pltpu.VMEM((K,), jnp.int32),
        pltpu.VMEM((K,), jnp.int32),
        pltpu.VMEM((K, 128), jnp.float32),
        pltpu.VMEM((K,), jnp.int32),
        pltpu.VMEM((K,), jnp.int32),
        pltpu.VMEM((K, 128), jnp.float32),
    ]
    if with_deg:
        out_type.append(jax.ShapeDtypeStruct((NC, NPAD), jnp.float32))
        scratch.append(pltpu.VMEM((K,), jnp.float32))
    scratch.extend([pltpu.SemaphoreType.DMA] * 4)
    scratch.append(pltpu.VMEM_SHARED((NPAD, 128), jnp.float32))
    if with_deg:
        scratch.append(pltpu.VMEM_SHARED((NPAD,), jnp.float32))
    mesh = plsc.VectorSubcoreMesh(core_axis_name="c", subcore_axis_name="s",
                                  num_cores=NC, num_subcores=NS)
    return pl.kernel(
        functools.partial(_sc_body, with_deg),
        out_type=tuple(out_type),
        mesh=mesh,
        scratch_types=tuple(scratch),
        name="sage_segsum_sc" + ("_deg" if with_deg else ""),
    )


def _dense_body(a0, a1, d0, d1, xb, WlT, bl, WrT, out):
    deg = jnp.maximum(d0[...] + d1[...], 1.0)
    agg = (a0[...] + a1[...]) / deg
    h = (jnp.dot(agg, WlT[...], preferred_element_type=jnp.float32)
         + bl[...]
         + jnp.dot(xb[...], WrT[...], preferred_element_type=jnp.float32))
    out[...] = jnp.maximum(h, 0.0)


BR = 1280


def _dense_call(a0, a1, d0, d1, xb, WlT, bl, WrT):
    nblk = NPAD // BR
    row = lambda i: (i, 0)
    fixed = lambda i: (0, 0)
    return pl.pallas_call(
        _dense_body,
        grid=(nblk,),
        in_specs=[
            pl.BlockSpec((BR, 128), row),
            pl.BlockSpec((BR, 128), row),
            pl.BlockSpec((BR, 1), row),
            pl.BlockSpec((BR, 1), row),
            pl.BlockSpec((BR, 128), row),
            pl.BlockSpec((128, 128), fixed),
            pl.BlockSpec((1, 128), fixed),
            pl.BlockSpec((128, 128), fixed),
        ],
        out_specs=pl.BlockSpec((BR, 128), row),
        out_shape=jax.ShapeDtypeStruct((NPAD, 128), jnp.float32),
    )(a0, a1, d0, d1, xb, WlT, bl, WrT)


def _tail_body(h2, batch2d, set01, WmdT, WmmT, WmxT, bm, W1T, b1,
               W2Tp, b2p, out):
    b = batch2d[...]
    g3 = lax.broadcasted_iota(jnp.int32, (128, 80, 128), 0)
    cmp = (b[None, :, :] < g3).astype(jnp.int32)
    base = jnp.sum(jnp.sum(cmp, axis=2), axis=1, keepdims=True)
    idx0 = jnp.clip(base + set01[:, 0:1], 0, N - 1)
    idx1 = jnp.clip(base + set01[:, 1:2], 0, N - 1)
    col = lax.broadcasted_iota(jnp.int32, (128, NPAD), 1)
    h = h2[...]
    xs0 = jnp.dot((col == idx0).astype(jnp.float32), h,
                  preferred_element_type=jnp.float32)
    xs1 = jnp.dot((col == idx1).astype(jnp.float32), h,
                  preferred_element_type=jnp.float32)
    d = jnp.abs(xs0 - xs1)
    m = (xs0 + xs1) * 0.5
    x = jnp.maximum(xs0, xs1)
    pooled = (jnp.dot(d, WmdT[...], preferred_element_type=jnp.float32)
              + jnp.dot(m, WmmT[...], preferred_element_type=jnp.float32)
              + jnp.dot(x, WmxT[...], preferred_element_type=jnp.float32)
              + bm[...])
    f = jnp.maximum(
        jnp.dot(pooled, W1T[...], preferred_element_type=jnp.float32) + b1[...],
        0.0)
    logits = jnp.dot(f, W2Tp[...], preferred_element_type=jnp.float32) + b2p[...]
    mx = jnp.max(logits, axis=1, keepdims=True)
    lse = jnp.log(jnp.sum(jnp.exp(logits - mx), axis=1, keepdims=True))
    out[...] = logits - mx - lse


def _tail_call(h2, batch2d, set01, WmdT, WmmT, WmxT, bm, W1T, b1, W2Tp, b2p):
    return pl.pallas_call(
        _tail_body,
        out_shape=jax.ShapeDtypeStruct((128, 128), jnp.float32),
    )(h2, batch2d, set01, WmdT, WmmT, WmxT, bm, W1T, b1, W2Tp, b2p)


def kernel(x, edge_index, set_indices, batch, num_graphs,
           Wl1, bl1, Wr1, Wl2, bl2, Wr2, Wm, bm, W1, b1, W2, b2):
    del num_graphs
    f32 = jnp.float32

    xp = jnp.pad(x, ((0, NPAD - N), (0, 0)))
    pad_iota = jnp.arange(EPAD - E, dtype=jnp.int32)
    src2d = jnp.concatenate(
        [edge_index[0], pad_iota % N]).reshape(NCHUNK, K)
    dst2d = jnp.concatenate(
        [edge_index[1], N + pad_iota % (NPAD - N)]).reshape(NCHUNK, K)
    zrow = jnp.zeros((ROWS_PER_SUB, 128), f32)
    zone = jnp.zeros((ROWS_PER_SUB,), f32)
    batch2d = jnp.pad(batch, (0, NPAD - N), constant_values=127).reshape(80, 128)
    set01 = jnp.pad(set_indices, ((0, 128 - set_indices.shape[0]), (0, 6)))
    Wl1T, Wr1T = Wl1.T, Wr1.T
    Wl2T, Wr2T = Wl2.T, Wr2.T
    bl1r, bl2r = bl1.reshape(1, 128), bl2.reshape(1, 128)
    WmdT = Wm[:, 0:128].T
    WmmT = Wm[:, 128:256].T
    WmxT = Wm[:, 256:384].T
    bmr = bm.reshape(1, 128)
    W1T = W1.T
    b1r = b1.reshape(1, 128)
    W2Tp = jnp.pad(W2.T, ((0, 0), (0, 128 - W2.shape[0])))
    b2p = jnp.pad(b2, (0, 128 - W2.shape[0]),
                  constant_values=-1e30).reshape(1, 128)

    acc1, deg = _make_sc_call(True)(xp, src2d, dst2d, zrow, zone)
    d0 = deg[0].reshape(NPAD, 1)
    d1 = deg[1].reshape(NPAD, 1)
    h1 = _dense_call(acc1[0], acc1[1], d0, d1, xp, Wl1T, bl1r, Wr1T)

    acc2 = _make_sc_call(False)(h1, src2d, dst2d, zrow, zone)[0]
    h2 = _dense_call(acc2[0], acc2[1], d0, d1, h1, Wl2T, bl2r, Wr2T)

    outp = _tail_call(h2, batch2d, set01, WmdT, WmmT, WmxT, bmr,
                      W1T, b1r, W2Tp, b2p)
    return outp[:set_indices.shape[0], :W2.shape[0]]


# speedup vs baseline: 3.2640x; 1.3521x over previous
import functools
import jax
import jax.numpy as jnp
from jax import lax
from jax.experimental import pallas as pl
from jax.experimental.pallas import tpu as pltpu
from jax.experimental.pallas import tpu_sc as plsc

N = 10000
NPAD = 10240
E = 320000
K = 128
NC, NS = 2, 16
NW = NC * NS
CPW = 80
HB = 40
NCHUNK = NW * CPW
EPAD = NCHUNK * K
ROWS_PER_SUB = NPAD // NS


def _sc_body(with_deg, x_hbm, src_hbm, dst_hbm, zrow_hbm, zone_hbm,
             *refs):
    if with_deg:
        (acc_out, deg_out, src_l, dst_l, rows0, rows1,
         ones_v, semg0, sems0, semg1, sems1, acc_sh, deg_sh) = refs
    else:
        (acc_out, src_l, dst_l, rows0, rows1,
         semg0, sems0, semg1, sems1, acc_sh) = refs
    c = lax.axis_index("c")
    s = lax.axis_index("s")
    w = s * NC + c

    pltpu.sync_copy(zrow_hbm, acc_sh.at[pl.ds(s * ROWS_PER_SUB, ROWS_PER_SUB)])
    if with_deg:
        pltpu.sync_copy(zone_hbm, deg_sh.at[pl.ds(s * ROWS_PER_SUB, ROWS_PER_SUB)])
        for j in range(K // 16):
            ones_v[pl.ds(j * 16, 16)] = jnp.ones((16,), jnp.float32)
    plsc.subcore_barrier()

    bufs = ((rows0, semg0, sems0), (rows1, semg1, sems1))

    def gather(t, b):
        return pltpu.make_async_copy(x_hbm.at[src_l.at[t]], bufs[b][0],
                                     bufs[b][1])

    def scatter_start(t, b):
        pltpu.async_copy(bufs[b][0], acc_sh.at[dst_l.at[t]], bufs[b][2],
                         add=True)
        if with_deg:
            pltpu.sync_copy(ones_v, deg_sh.at[dst_l.at[t]], add=True)

    def scatter_wait(t, b):
        pltpu.make_async_copy(bufs[b][0], acc_sh.at[dst_l.at[t]],
                              bufs[b][2]).wait()

    def step(tcur, bcur, tnext):
        bnext = 1 - bcur
        gather(tcur, bcur).wait()
        scatter_start(tcur, bcur)

        @pl.when(tnext < HB)
        def _():
            @pl.when(tnext - 2 >= 0)
            def _():
                scatter_wait(tnext - 2, bnext)
            gather(tnext, bnext).start()

    def pair(i, carry):
        t0 = 2 * i
        step(t0, 0, t0 + 1)
        step(t0 + 1, 1, t0 + 2)
        return carry

    for h in range(CPW // HB):
        base = w * CPW + h * HB
        pltpu.sync_copy(src_hbm.at[pl.ds(base, HB)], src_l)
        pltpu.sync_copy(dst_hbm.at[pl.ds(base, HB)], dst_l)
        gather(0, 0).start()
        lax.fori_loop(0, HB // 2, pair, 0)
        scatter_wait(HB - 2, 0)
        scatter_wait(HB - 1, 1)
    plsc.subcore_barrier()

    sl = pl.ds(s * ROWS_PER_SUB, ROWS_PER_SUB)
    pltpu.sync_copy(acc_sh.at[sl], acc_out.at[c, sl])
    if with_deg:
        pltpu.sync_copy(deg_sh.at[sl], deg_out.at[c, sl])


def _make_sc_call(with_deg):
    out_type = [jax.ShapeDtypeStruct((NC, NPAD, 128), jnp.float32)]
    scratch = [
        pltpu.VMEM((HB, K), jnp.int32),
        pltpu.VMEM((HB, K), jnp.int32),
        pltpu.VMEM((K, 128), jnp.float32),
        pltpu.VMEM((K, 128), jnp.float32),
    ]
    if with_deg:
        out_type.append(jax.ShapeDtypeStruct((NC, NPAD), jnp.float32))
        scratch.append(pltpu.VMEM((K,), jnp.float32))
    scratch.extend([pltpu.SemaphoreType.DMA] * 4)
    scratch.append(pltpu.VMEM_SHARED((NPAD, 128), jnp.float32))
    if with_deg:
        scratch.append(pltpu.VMEM_SHARED((NPAD,), jnp.float32))
    mesh = plsc.VectorSubcoreMesh(core_axis_name="c", subcore_axis_name="s",
                                  num_cores=NC, num_subcores=NS)
    return pl.kernel(
        functools.partial(_sc_body, with_deg),
        out_type=tuple(out_type),
        mesh=mesh,
        scratch_types=tuple(scratch),
        name="sage_segsum_sc" + ("_deg" if with_deg else ""),
    )


def _dense_body(a0, a1, d0, d1, xb, WlT, bl, WrT, out):
    deg = jnp.maximum(d0[...] + d1[...], 1.0)
    agg = (a0[...] + a1[...]) / deg
    h = (jnp.dot(agg, WlT[...], preferred_element_type=jnp.float32)
         + bl[...]
         + jnp.dot(xb[...], WrT[...], preferred_element_type=jnp.float32))
    out[...] = jnp.maximum(h, 0.0)


BR = 1280


def _dense_call(a0, a1, d0, d1, xb, WlT, bl, WrT):
    nblk = NPAD // BR
    row = lambda i: (i, 0)
    fixed = lambda i: (0, 0)
    return pl.pallas_call(
        _dense_body,
        grid=(nblk,),
        in_specs=[
            pl.BlockSpec((BR, 128), row),
            pl.BlockSpec((BR, 128), row),
            pl.BlockSpec((BR, 1), row),
            pl.BlockSpec((BR, 1), row),
            pl.BlockSpec((BR, 128), row),
            pl.BlockSpec((128, 128), fixed),
            pl.BlockSpec((1, 128), fixed),
            pl.BlockSpec((128, 128), fixed),
        ],
        out_specs=pl.BlockSpec((BR, 128), row),
        out_shape=jax.ShapeDtypeStruct((NPAD, 128), jnp.float32),
    )(a0, a1, d0, d1, xb, WlT, bl, WrT)


def _tail_body(h2, batch2d, set01, WmdT, WmmT, WmxT, bm, W1T, b1,
               W2Tp, b2p, out):
    b = batch2d[...]
    g3 = lax.broadcasted_iota(jnp.int32, (128, 80, 128), 0)
    cmp = (b[None, :, :] < g3).astype(jnp.int32)
    base = jnp.sum(jnp.sum(cmp, axis=2), axis=1, keepdims=True)
    idx0 = jnp.clip(base + set01[:, 0:1], 0, N - 1)
    idx1 = jnp.clip(base + set01[:, 1:2], 0, N - 1)
    col = lax.broadcasted_iota(jnp.int32, (128, NPAD), 1)
    h = h2[...]
    xs0 = jnp.dot((col == idx0).astype(jnp.float32), h,
                  preferred_element_type=jnp.float32)
    xs1 = jnp.dot((col == idx1).astype(jnp.float32), h,
                  preferred_element_type=jnp.float32)
    d = jnp.abs(xs0 - xs1)
    m = (xs0 + xs1) * 0.5
    x = jnp.maximum(xs0, xs1)
    pooled = (jnp.dot(d, WmdT[...], preferred_element_type=jnp.float32)
              + jnp.dot(m, WmmT[...], preferred_element_type=jnp.float32)
              + jnp.dot(x, WmxT[...], preferred_element_type=jnp.float32)
              + bm[...])
    f = jnp.maximum(
        jnp.dot(pooled, W1T[...], preferred_element_type=jnp.float32) + b1[...],
        0.0)
    logits = jnp.dot(f, W2Tp[...], preferred_element_type=jnp.float32) + b2p[...]
    mx = jnp.max(logits, axis=1, keepdims=True)
    lse = jnp.log(jnp.sum(jnp.exp(logits - mx), axis=1, keepdims=True))
    out[...] = logits - mx - lse


def _tail_call(h2, batch2d, set01, WmdT, WmmT, WmxT, bm, W1T, b1, W2Tp, b2p):
    return pl.pallas_call(
        _tail_body,
        out_shape=jax.ShapeDtypeStruct((128, 128), jnp.float32),
    )(h2, batch2d, set01, WmdT, WmmT, WmxT, bm, W1T, b1, W2Tp, b2p)


def kernel(x, edge_index, set_indices, batch, num_graphs,
           Wl1, bl1, Wr1, Wl2, bl2, Wr2, Wm, bm, W1, b1, W2, b2):
    del num_graphs
    f32 = jnp.float32

    xp = jnp.pad(x, ((0, NPAD - N), (0, 0)))
    pad_iota = jnp.arange(EPAD - E, dtype=jnp.int32)
    src2d = jnp.concatenate(
        [edge_index[0], pad_iota % N]).reshape(NCHUNK, K)
    dst2d = jnp.concatenate(
        [edge_index[1], N + pad_iota % (NPAD - N)]).reshape(NCHUNK, K)
    zrow = jnp.zeros((ROWS_PER_SUB, 128), f32)
    zone = jnp.zeros((ROWS_PER_SUB,), f32)
    batch2d = jnp.pad(batch, (0, NPAD - N), constant_values=127).reshape(80, 128)
    set01 = jnp.pad(set_indices, ((0, 128 - set_indices.shape[0]), (0, 6)))
    Wl1T, Wr1T = Wl1.T, Wr1.T
    Wl2T, Wr2T = Wl2.T, Wr2.T
    bl1r, bl2r = bl1.reshape(1, 128), bl2.reshape(1, 128)
    WmdT = Wm[:, 0:128].T
    WmmT = Wm[:, 128:256].T
    WmxT = Wm[:, 256:384].T
    bmr = bm.reshape(1, 128)
    W1T = W1.T
    b1r = b1.reshape(1, 128)
    W2Tp = jnp.pad(W2.T, ((0, 0), (0, 128 - W2.shape[0])))
    b2p = jnp.pad(b2, (0, 128 - W2.shape[0]),
                  constant_values=-1e30).reshape(1, 128)

    acc1, deg = _make_sc_call(True)(xp, src2d, dst2d, zrow, zone)
    d0 = deg[0].reshape(NPAD, 1)
    d1 = deg[1].reshape(NPAD, 1)
    h1 = _dense_call(acc1[0], acc1[1], d0, d1, xp, Wl1T, bl1r, Wr1T)

    acc2 = _make_sc_call(False)(h1, src2d, dst2d, zrow, zone)[0]
    h2 = _dense_call(acc2[0], acc2[1], d0, d1, h1, Wl2T, bl2r, Wr2T)

    outp = _tail_call(h2, batch2d, set01, WmdT, WmmT, WmxT, bmr,
                      W1T, b1r, W2Tp, b2p)
    return outp[:set_indices.shape[0], :W2.shape[0]]


# speedup vs baseline: 3.3173x; 1.0163x over previous
import functools
import jax
import jax.numpy as jnp
from jax import lax
from jax.experimental import pallas as pl
from jax.experimental.pallas import tpu as pltpu
from jax.experimental.pallas import tpu_sc as plsc

N = 10000
NPAD = 10240
E = 320000
K = 128
NC, NS = 2, 16
NW = NC * NS
CPW = 80
HB = 40
NCHUNK = NW * CPW
EPAD = NCHUNK * K
ROWS_PER_SUB = NPAD // NS


def _sc_body(with_deg, x_hbm, src_hbm, dst_hbm, zrow_hbm, zone_hbm,
             *refs):
    if with_deg:
        (acc_out, deg_out, src_l, dst_l, rows0, rows1,
         ones_v, semg0, sems0, semg1, sems1, acc_sh, deg_sh) = refs
    else:
        (acc_out, src_l, dst_l, rows0, rows1,
         semg0, sems0, semg1, sems1, acc_sh) = refs
    c = lax.axis_index("c")
    s = lax.axis_index("s")
    w = s * NC + c

    pltpu.sync_copy(zrow_hbm, acc_sh.at[pl.ds(s * ROWS_PER_SUB, ROWS_PER_SUB)])
    if with_deg:
        pltpu.sync_copy(zone_hbm, deg_sh.at[pl.ds(s * ROWS_PER_SUB, ROWS_PER_SUB)])
        for j in range(K // 16):
            ones_v[pl.ds(j * 16, 16)] = jnp.ones((16,), jnp.float32)
    plsc.subcore_barrier()

    bufs = ((rows0, semg0, sems0), (rows1, semg1, sems1))

    def gather(t, b):
        return pltpu.make_async_copy(x_hbm.at[src_l.at[t]], bufs[b][0],
                                     bufs[b][1])

    def scatter_start(t, b):
        pltpu.async_copy(bufs[b][0], acc_sh.at[dst_l.at[t]], bufs[b][2],
                         add=True)
        if with_deg:
            pltpu.sync_copy(ones_v, deg_sh.at[dst_l.at[t]], add=True)

    def scatter_wait(t, b):
        pltpu.make_async_copy(bufs[b][0], acc_sh.at[dst_l.at[t]],
                              bufs[b][2]).wait()

    def step(tcur, bcur, tnext):
        bnext = 1 - bcur
        gather(tcur, bcur).wait()
        scatter_start(tcur, bcur)

        @pl.when(tnext < HB)
        def _():
            @pl.when(tnext - 2 >= 0)
            def _():
                scatter_wait(tnext - 2, bnext)
            gather(tnext, bnext).start()

    def pair(i, carry):
        t0 = 2 * i
        step(t0, 0, t0 + 1)
        step(t0 + 1, 1, t0 + 2)
        return carry

    for h in range(CPW // HB):
        base = w * CPW + h * HB
        pltpu.sync_copy(src_hbm.at[pl.ds(base, HB)], src_l)
        pltpu.sync_copy(dst_hbm.at[pl.ds(base, HB)], dst_l)
        gather(0, 0).start()
        lax.fori_loop(0, HB // 2, pair, 0)
        scatter_wait(HB - 2, 0)
        scatter_wait(HB - 1, 1)
    plsc.subcore_barrier()

    sl = pl.ds(s * ROWS_PER_SUB, ROWS_PER_SUB)
    pltpu.sync_copy(acc_sh.at[sl], acc_out.at[c, sl])
    if with_deg:
        pltpu.sync_copy(deg_sh.at[sl], deg_out.at[c, sl])


def _make_sc_call(with_deg):
    out_type = [jax.ShapeDtypeStruct((NC, NPAD, 128), jnp.float32)]
    scratch = [
        pltpu.VMEM((HB, K), jnp.int32),
        pltpu.VMEM((HB, K), jnp.int32),
        pltpu.VMEM((K, 128), jnp.float32),
        pltpu.VMEM((K, 128), jnp.float32),
    ]
    if with_deg:
        out_type.append(jax.ShapeDtypeStruct((NC, NPAD), jnp.float32))
        scratch.append(pltpu.VMEM((K,), jnp.float32))
    scratch.extend([pltpu.SemaphoreType.DMA] * 4)
    scratch.append(pltpu.VMEM_SHARED((NPAD, 128), jnp.float32))
    if with_deg:
        scratch.append(pltpu.VMEM_SHARED((NPAD,), jnp.float32))
    mesh = plsc.VectorSubcoreMesh(core_axis_name="c", subcore_axis_name="s",
                                  num_cores=NC, num_subcores=NS)
    return pl.kernel(
        functools.partial(_sc_body, with_deg),
        out_type=tuple(out_type),
        mesh=mesh,
        scratch_types=tuple(scratch),
        name="sage_segsum_sc" + ("_deg" if with_deg else ""),
    )


def _dense_body(a0, a1, d0, d1, xb, WlT, bl, WrT, out):
    deg = jnp.maximum(d0[...] + d1[...], 1.0)
    agg = (a0[...] + a1[...]) / deg
    h = (jnp.dot(agg, WlT[...], preferred_element_type=jnp.float32)
         + bl[...]
         + jnp.dot(xb[...], WrT[...], preferred_element_type=jnp.float32))
    out[...] = jnp.maximum(h, 0.0)


BR = 1280


def _dense_call(a0, a1, d0, d1, xb, WlT, bl, WrT):
    nblk = NPAD // BR
    row = lambda i: (i, 0)
    fixed = lambda i: (0, 0)
    return pl.pallas_call(
        _dense_body,
        grid=(nblk,),
        in_specs=[
            pl.BlockSpec((BR, 128), row),
            pl.BlockSpec((BR, 128), row),
            pl.BlockSpec((BR, 1), row),
            pl.BlockSpec((BR, 1), row),
            pl.BlockSpec((BR, 128), row),
            pl.BlockSpec((128, 128), fixed),
            pl.BlockSpec((1, 128), fixed),
            pl.BlockSpec((128, 128), fixed),
        ],
        out_specs=pl.BlockSpec((BR, 128), row),
        out_shape=jax.ShapeDtypeStruct((NPAD, 128), jnp.float32),
    )(a0, a1, d0, d1, xb, WlT, bl, WrT)


def _tail_body(a0, a1, d0, d1, xb, WlT, bl, WrT,
               batch2d, set01, WmdT, WmmT, WmxT, bm, W1T, b1,
               W2Tp, b2p, out):
    deg = jnp.maximum(d0[...] + d1[...], 1.0)
    agg = (a0[...] + a1[...]) / deg
    h2 = jnp.maximum(
        jnp.dot(agg, WlT[...], preferred_element_type=jnp.float32)
        + bl[...]
        + jnp.dot(xb[...], WrT[...], preferred_element_type=jnp.float32),
        0.0)
    b = batch2d[...]
    g3 = lax.broadcasted_iota(jnp.int32, (128, 80, 128), 0)
    cmp = (b[None, :, :] < g3).astype(jnp.int32)
    base = jnp.sum(jnp.sum(cmp, axis=2), axis=1, keepdims=True)
    idx0 = jnp.clip(base + set01[:, 0:1], 0, N - 1)
    idx1 = jnp.clip(base + set01[:, 1:2], 0, N - 1)
    col = lax.broadcasted_iota(jnp.int32, (128, NPAD), 1)
    xs0 = jnp.dot((col == idx0).astype(jnp.float32), h2,
                  preferred_element_type=jnp.float32)
    xs1 = jnp.dot((col == idx1).astype(jnp.float32), h2,
                  preferred_element_type=jnp.float32)
    d = jnp.abs(xs0 - xs1)
    m = (xs0 + xs1) * 0.5
    x = jnp.maximum(xs0, xs1)
    pooled = (jnp.dot(d, WmdT[...], preferred_element_type=jnp.float32)
              + jnp.dot(m, WmmT[...], preferred_element_type=jnp.float32)
              + jnp.dot(x, WmxT[...], preferred_element_type=jnp.float32)
              + bm[...])
    f = jnp.maximum(
        jnp.dot(pooled, W1T[...], preferred_element_type=jnp.float32) + b1[...],
        0.0)
    logits = jnp.dot(f, W2Tp[...], preferred_element_type=jnp.float32) + b2p[...]
    mx = jnp.max(logits, axis=1, keepdims=True)
    lse = jnp.log(jnp.sum(jnp.exp(logits - mx), axis=1, keepdims=True))
    out[...] = logits - mx - lse


def _tail_call(a0, a1, d0, d1, xb, WlT, bl, WrT,
               batch2d, set01, WmdT, WmmT, WmxT, bm, W1T, b1, W2Tp, b2p):
    return pl.pallas_call(
        _tail_body,
        out_shape=jax.ShapeDtypeStruct((128, 128), jnp.float32),
    )(a0, a1, d0, d1, xb, WlT, bl, WrT,
      batch2d, set01, WmdT, WmmT, WmxT, bm, W1T, b1, W2Tp, b2p)


def kernel(x, edge_index, set_indices, batch, num_graphs,
           Wl1, bl1, Wr1, Wl2, bl2, Wr2, Wm, bm, W1, b1, W2, b2):
    del num_graphs
    f32 = jnp.float32

    xp = jnp.pad(x, ((0, NPAD - N), (0, 0)))
    pad_iota = jnp.arange(EPAD - E, dtype=jnp.int32)
    src2d = jnp.concatenate(
        [edge_index[0], pad_iota % N]).reshape(NCHUNK, K)
    dst2d = jnp.concatenate(
        [edge_index[1], N + pad_iota % (NPAD - N)]).reshape(NCHUNK, K)
    zrow = jnp.zeros((ROWS_PER_SUB, 128), f32)
    zone = jnp.zeros((ROWS_PER_SUB,), f32)
    batch2d = jnp.pad(batch, (0, NPAD - N), constant_values=127).reshape(80, 128)
    set01 = jnp.pad(set_indices, ((0, 128 - set_indices.shape[0]), (0, 6)))
    Wl1T, Wr1T = Wl1.T, Wr1.T
    Wl2T, Wr2T = Wl2.T, Wr2.T
    bl1r, bl2r = bl1.reshape(1, 128), bl2.reshape(1, 128)
    WmdT = Wm[:, 0:128].T
    WmmT = Wm[:, 128:256].T
    WmxT = Wm[:, 256:384].T
    bmr = bm.reshape(1, 128)
    W1T = W1.T
    b1r = b1.reshape(1, 128)
    W2Tp = jnp.pad(W2.T, ((0, 0), (0, 128 - W2.shape[0])))
    b2p = jnp.pad(b2, (0, 128 - W2.shape[0]),
                  constant_values=-1e30).reshape(1, 128)

    acc1, deg = _make_sc_call(True)(xp, src2d, dst2d, zrow, zone)
    d0 = deg[0].reshape(NPAD, 1)
    d1 = deg[1].reshape(NPAD, 1)
    h1 = _dense_call(acc1[0], acc1[1], d0, d1, xp, Wl1T, bl1r, Wr1T)

    acc2 = _make_sc_call(False)(h1, src2d, dst2d, zrow, zone)[0]
    outp = _tail_call(acc2[0], acc2[1], d0, d1, h1, Wl2T, bl2r, Wr2T,
                      batch2d, set01, WmdT, WmmT, WmxT, bmr,
                      W1T, b1r, W2Tp, b2p)
    return outp[:set_indices.shape[0], :W2.shape[0]]


# speedup vs baseline: 3.3990x; 1.0246x over previous
import functools
import jax
import jax.numpy as jnp
from jax import lax
from jax.experimental import pallas as pl
from jax.experimental.pallas import tpu as pltpu
from jax.experimental.pallas import tpu_sc as plsc

N = 10000
NPAD = 10240
E = 320000
K = 128
NC, NS = 2, 16
NW = NC * NS
CPW = 80
HB = 40
NCHUNK = NW * CPW
EPAD = NCHUNK * K
ROWS_PER_SUB = NPAD // NS


def _sc_body(with_deg, x_hbm, src_hbm, dst_hbm, zrow_hbm, zone_hbm,
             *refs):
    if with_deg:
        (acc_out, deg_out, src_l, dst_l, rows0, rows1,
         ones_v, semg0, sems0, semg1, sems1, semd, acc_sh, deg_sh) = refs
    else:
        (acc_out, src_l, dst_l, rows0, rows1,
         semg0, sems0, semg1, sems1, acc_sh) = refs
    c = lax.axis_index("c")
    s = lax.axis_index("s")
    w = s * NC + c

    pltpu.sync_copy(zrow_hbm, acc_sh.at[pl.ds(s * ROWS_PER_SUB, ROWS_PER_SUB)])
    if with_deg:
        pltpu.sync_copy(zone_hbm, deg_sh.at[pl.ds(s * ROWS_PER_SUB, ROWS_PER_SUB)])
        for j in range(K // 16):
            ones_v[pl.ds(j * 16, 16)] = jnp.ones((16,), jnp.float32)
    plsc.subcore_barrier()

    bufs = ((rows0, semg0, sems0), (rows1, semg1, sems1))

    def gather(t, b):
        return pltpu.make_async_copy(x_hbm.at[src_l.at[t]], bufs[b][0],
                                     bufs[b][1])

    def scatter_start(t, b):
        pltpu.async_copy(bufs[b][0], acc_sh.at[dst_l.at[t]], bufs[b][2],
                         add=True)
        if with_deg:
            pltpu.async_copy(ones_v, deg_sh.at[dst_l.at[t]], semd,
                             add=True)

            @pl.when(t - 2 >= 0)
            def _():
                pltpu.make_async_copy(ones_v, deg_sh.at[dst_l.at[t - 2]],
                                      semd).wait()

    def scatter_wait(t, b):
        pltpu.make_async_copy(bufs[b][0], acc_sh.at[dst_l.at[t]],
                              bufs[b][2]).wait()
        if with_deg:
            @pl.when(t + 2 >= HB)
            def _():
                pltpu.make_async_copy(ones_v, deg_sh.at[dst_l.at[t]],
                                      semd).wait()

    def step(tcur, bcur, tnext):
        bnext = 1 - bcur
        gather(tcur, bcur).wait()
        scatter_start(tcur, bcur)

        @pl.when(tnext < HB)
        def _():
            @pl.when(tnext - 2 >= 0)
            def _():
                scatter_wait(tnext - 2, bnext)
            gather(tnext, bnext).start()

    def pair(i, carry):
        t0 = 2 * i
        step(t0, 0, t0 + 1)
        step(t0 + 1, 1, t0 + 2)
        return carry

    for h in range(CPW // HB):
        base = w * CPW + h * HB
        pltpu.sync_copy(src_hbm.at[pl.ds(base, HB)], src_l)
        pltpu.sync_copy(dst_hbm.at[pl.ds(base, HB)], dst_l)
        gather(0, 0).start()
        lax.fori_loop(0, HB // 2, pair, 0)
        scatter_wait(HB - 2, 0)
        scatter_wait(HB - 1, 1)
    plsc.subcore_barrier()

    sl = pl.ds(s * ROWS_PER_SUB, ROWS_PER_SUB)
    pltpu.sync_copy(acc_sh.at[sl], acc_out.at[c, sl])
    if with_deg:
        pltpu.sync_copy(deg_sh.at[sl], deg_out.at[c, sl])


def _make_sc_call(with_deg):
    out_type = [jax.ShapeDtypeStruct((NC, NPAD, 128), jnp.float32)]
    scratch = [
        pltpu.VMEM((HB, K), jnp.int32),
        pltpu.VMEM((HB, K), jnp.int32),
        pltpu.VMEM((K, 128), jnp.float32),
        pltpu.VMEM((K, 128), jnp.float32),
    ]
    if with_deg:
        out_type.append(jax.ShapeDtypeStruct((NC, NPAD), jnp.float32))
        scratch.append(pltpu.VMEM((K,), jnp.float32))
    scratch.extend([pltpu.SemaphoreType.DMA] * (5 if with_deg else 4))
    scratch.append(pltpu.VMEM_SHARED((NPAD, 128), jnp.float32))
    if with_deg:
        scratch.append(pltpu.VMEM_SHARED((NPAD,), jnp.float32))
    mesh = plsc.VectorSubcoreMesh(core_axis_name="c", subcore_axis_name="s",
                                  num_cores=NC, num_subcores=NS)
    return pl.kernel(
        functools.partial(_sc_body, with_deg),
        out_type=tuple(out_type),
        mesh=mesh,
        scratch_types=tuple(scratch),
        name="sage_segsum_sc" + ("_deg" if with_deg else ""),
    )


def _dense_body(a0, a1, d0, d1, xb, WlT, bl, WrT, out):
    deg = jnp.maximum(d0[...] + d1[...], 1.0)
    agg = (a0[...] + a1[...]) / deg
    h = (jnp.dot(agg, WlT[...], preferred_element_type=jnp.float32)
         + bl[...]
         + jnp.dot(xb[...], WrT[...], preferred_element_type=jnp.float32))
    out[...] = jnp.maximum(h, 0.0)


BR = 1280


def _dense_call(a0, a1, d0, d1, xb, WlT, bl, WrT):
    nblk = NPAD // BR
    row = lambda i: (i, 0)
    fixed = lambda i: (0, 0)
    return pl.pallas_call(
        _dense_body,
        grid=(nblk,),
        in_specs=[
            pl.BlockSpec((BR, 128), row),
            pl.BlockSpec((BR, 128), row),
            pl.BlockSpec((BR, 1), row),
            pl.BlockSpec((BR, 1), row),
            pl.BlockSpec((BR, 128), row),
            pl.BlockSpec((128, 128), fixed),
            pl.BlockSpec((1, 128), fixed),
            pl.BlockSpec((128, 128), fixed),
        ],
        out_specs=pl.BlockSpec((BR, 128), row),
        out_shape=jax.ShapeDtypeStruct((NPAD, 128), jnp.float32),
    )(a0, a1, d0, d1, xb, WlT, bl, WrT)


def _tail_body(a0, a1, d0, d1, xb, WlT, bl, WrT,
               batch2d, set01, WmdT, WmmT, WmxT, bm, W1T, b1,
               W2Tp, b2p, out):
    deg = jnp.maximum(d0[...] + d1[...], 1.0)
    agg = (a0[...] + a1[...]) / deg
    h2 = jnp.maximum(
        jnp.dot(agg, WlT[...], preferred_element_type=jnp.float32)
        + bl[...]
        + jnp.dot(xb[...], WrT[...], preferred_element_type=jnp.float32),
        0.0)
    b = batch2d[...]
    g3 = lax.broadcasted_iota(jnp.int32, (128, 80, 128), 0)
    cmp = (b[None, :, :] < g3).astype(jnp.int32)
    base = jnp.sum(jnp.sum(cmp, axis=2), axis=1, keepdims=True)
    idx0 = jnp.clip(base + set01[:, 0:1], 0, N - 1)
    idx1 = jnp.clip(base + set01[:, 1:2], 0, N - 1)
    col = lax.broadcasted_iota(jnp.int32, (128, NPAD), 1)
    xs0 = jnp.dot((col == idx0).astype(jnp.float32), h2,
                  preferred_element_type=jnp.float32)
    xs1 = jnp.dot((col == idx1).astype(jnp.float32), h2,
                  preferred_element_type=jnp.float32)
    d = jnp.abs(xs0 - xs1)
    m = (xs0 + xs1) * 0.5
    x = jnp.maximum(xs0, xs1)
    pooled = (jnp.dot(d, WmdT[...], preferred_element_type=jnp.float32)
              + jnp.dot(m, WmmT[...], preferred_element_type=jnp.float32)
              + jnp.dot(x, WmxT[...], preferred_element_type=jnp.float32)
              + bm[...])
    f = jnp.maximum(
        jnp.dot(pooled, W1T[...], preferred_element_type=jnp.float32) + b1[...],
        0.0)
    logits = jnp.dot(f, W2Tp[...], preferred_element_type=jnp.float32) + b2p[...]
    mx = jnp.max(logits, axis=1, keepdims=True)
    lse = jnp.log(jnp.sum(jnp.exp(logits - mx), axis=1, keepdims=True))
    out[...] = logits - mx - lse


def _tail_call(a0, a1, d0, d1, xb, WlT, bl, WrT,
               batch2d, set01, WmdT, WmmT, WmxT, bm, W1T, b1, W2Tp, b2p):
    return pl.pallas_call(
        _tail_body,
        out_shape=jax.ShapeDtypeStruct((128, 128), jnp.float32),
    )(a0, a1, d0, d1, xb, WlT, bl, WrT,
      batch2d, set01, WmdT, WmmT, WmxT, bm, W1T, b1, W2Tp, b2p)


def kernel(x, edge_index, set_indices, batch, num_graphs,
           Wl1, bl1, Wr1, Wl2, bl2, Wr2, Wm, bm, W1, b1, W2, b2):
    del num_graphs
    f32 = jnp.float32

    xp = jnp.pad(x, ((0, NPAD - N), (0, 0)))
    pad_iota = jnp.arange(EPAD - E, dtype=jnp.int32)
    src2d = jnp.concatenate(
        [edge_index[0], pad_iota % N]).reshape(NCHUNK, K)
    dst2d = jnp.concatenate(
        [edge_index[1], N + pad_iota % (NPAD - N)]).reshape(NCHUNK, K)
    zrow = jnp.zeros((ROWS_PER_SUB, 128), f32)
    zone = jnp.zeros((ROWS_PER_SUB,), f32)
    batch2d = jnp.pad(batch, (0, NPAD - N), constant_values=127).reshape(80, 128)
    set01 = jnp.pad(set_indices, ((0, 128 - set_indices.shape[0]), (0, 6)))
    Wl1T, Wr1T = Wl1.T, Wr1.T
    Wl2T, Wr2T = Wl2.T, Wr2.T
    bl1r, bl2r = bl1.reshape(1, 128), bl2.reshape(1, 128)
    WmdT = Wm[:, 0:128].T
    WmmT = Wm[:, 128:256].T
    WmxT = Wm[:, 256:384].T
    bmr = bm.reshape(1, 128)
    W1T = W1.T
    b1r = b1.reshape(1, 128)
    W2Tp = jnp.pad(W2.T, ((0, 0), (0, 128 - W2.shape[0])))
    b2p = jnp.pad(b2, (0, 128 - W2.shape[0]),
                  constant_values=-1e30).reshape(1, 128)

    acc1, deg = _make_sc_call(True)(xp, src2d, dst2d, zrow, zone)
    d0 = deg[0].reshape(NPAD, 1)
    d1 = deg[1].reshape(NPAD, 1)
    h1 = _dense_call(acc1[0], acc1[1], d0, d1, xp, Wl1T, bl1r, Wr1T)

    acc2 = _make_sc_call(False)(h1, src2d, dst2d, zrow, zone)[0]
    outp = _tail_call(acc2[0], acc2[1], d0, d1, h1, Wl2T, bl2r, Wr2T,
                      batch2d, set01, WmdT, WmmT, WmxT, bmr,
                      W1T, b1r, W2Tp, b2p)
    return outp[:set_indices.shape[0], :W2.shape[0]]
